# Initial kernel scaffold; baseline (speedup 1.0000x reference)
import functools

import jax
import jax.numpy as jnp
from jax import lax
from jax.experimental import pallas as pl
from jax.experimental.pallas import tpu as pltpu
from jax.experimental.pallas import tpu_sc as plsc

N = 4096
E = 131072
IN_C = 128
OUT_C = 32
PAD = 48
NC = 2
NS = 16
NW = NC * NS
EPT = E // NW
CH = 128
NCHUNK = EPT // CH


def _tc_prep(x_ref, w_ref, a_ref, sup_ref, s2_ref):
    sup = jnp.dot(x_ref[...], w_ref[...], preferred_element_type=jnp.float32)
    sup_ref[...] = jnp.concatenate(
        [sup, jnp.zeros((N, PAD - OUT_C), jnp.float32)], axis=1)
    s2_ref[...] = lax.dot_general(
        sup, a_ref[...], (((1,), (1,)), ((), ())),
        preferred_element_type=jnp.float32)


def _sc_edges(sup_hbm, src_hbm, dst_hbm, adj_hbm, s2_hbm, out_hbm,
              acc_sh, stab_v, zbuf_v, idxs_v, idxd_v, adj_v, rows_v, w_v, sem):
    cid = lax.axis_index("c")
    sid = lax.axis_index("s")
    wid = sid * NC + cid

    zero16 = jnp.zeros((16,), jnp.float32)
    for r in range(CH):
        for k in range(PAD // 16):
            zbuf_v[r, 16 * k:16 * (k + 1)] = zero16
    rows_per_tile = N // NS
    for h in range(rows_per_tile // CH):
        pltpu.sync_copy(zbuf_v, acc_sh.at[pl.ds(sid * rows_per_tile + h * CH, CH)])
    pltpu.sync_copy(s2_hbm, stab_v)
    plsc.subcore_barrier()

    iota16 = lax.iota(jnp.int32, 16)
    zeros16 = jnp.zeros((16,), jnp.int32)
    ones16 = jnp.full((16,), 1, jnp.int32)
    col32 = jnp.full((16,), OUT_C, jnp.int32)

    def chunk(c, carry):
        base = wid * EPT + c * CH
        pltpu.sync_copy(src_hbm.at[pl.ds(base, CH)], idxs_v)
        pltpu.sync_copy(dst_hbm.at[pl.ds(base, CH)], idxd_v)
        pltpu.sync_copy(adj_hbm.at[pl.ds(base, CH)], adj_v)
        pltpu.async_copy(sup_hbm.at[idxd_v], rows_v, sem).wait()
        for g in range(CH // 16):
            sl = pl.ds(16 * g, 16)
            ss = plsc.load_gather(stab_v, [idxs_v[sl], zeros16])
            sd = plsc.load_gather(stab_v, [idxd_v[sl], ones16])
            s = ss + sd
            w = jnp.exp(jnp.maximum(s, 0.2 * s))
            w_v[sl] = w
            plsc.store_scatter(rows_v, [iota16 + 16 * g, col32], adj_v[sl])
        for e in range(CH):
            we = w_v[e]
            rows_v[e, 0:16] = rows_v[e, 0:16] * we
            rows_v[e, 16:32] = rows_v[e, 16:32] * we
        pltpu.sync_copy(rows_v, acc_sh.at[idxs_v], add=True)
        return carry

    lax.fori_loop(0, NCHUNK, chunk, 0)
    plsc.subcore_barrier()

    for h in range(rows_per_tile // CH):
        off = sid * rows_per_tile + h * CH
        pltpu.sync_copy(acc_sh.at[pl.ds(off, CH)],
                        out_hbm.at[cid, pl.ds(off, CH)])


def _tc_finish(parts_ref, out_ref):
    p = parts_ref[0] + parts_ref[1]
    r = p[:, OUT_C:OUT_C + 1]
    out_ref[...] = p[:, :OUT_C] / jnp.where(r == 0.0, 1.0, r)


@jax.jit
def kernel(x, edge_index, adj_values, weight, attention):
    attn2 = attention.reshape(2, OUT_C)
    sup_pad, s2 = pl.pallas_call(
        _tc_prep,
        out_shape=[
            jax.ShapeDtypeStruct((N, PAD), jnp.float32),
            jax.ShapeDtypeStruct((N, 2), jnp.float32),
        ],
    )(x, weight, attn2)

    src = edge_index[0]
    dst = edge_index[1]

    sc_call = functools.partial(
        pl.kernel,
        mesh=plsc.VectorSubcoreMesh(core_axis_name="c", subcore_axis_name="s"),
        out_type=jax.ShapeDtypeStruct((NC, N, PAD), jnp.float32),
        scratch_types=[
            pltpu.VMEM_SHARED((N, PAD), jnp.float32),
            pltpu.VMEM((N, 2), jnp.float32),
            pltpu.VMEM((CH, PAD), jnp.float32),
            pltpu.VMEM((CH,), jnp.int32),
            pltpu.VMEM((CH,), jnp.int32),
            pltpu.VMEM((CH,), jnp.float32),
            pltpu.VMEM((CH, PAD), jnp.float32),
            pltpu.VMEM((CH,), jnp.float32),
            pltpu.SemaphoreType.DMA,
        ],
    )(_sc_edges)
    parts = sc_call(sup_pad, src, dst, adj_values, s2)

    out = pl.pallas_call(
        _tc_finish,
        out_shape=jax.ShapeDtypeStruct((N, OUT_C), jnp.float32),
    )(parts)
    return out.reshape(N, 1, OUT_C)


# speedup vs baseline: 16.9842x; 16.9842x over previous
import functools

import jax
import jax.numpy as jnp
from jax import lax
from jax.experimental import pallas as pl
from jax.experimental.pallas import tpu as pltpu
from jax.experimental.pallas import tpu_sc as plsc

N = 4096
E = 131072
IN_C = 128
OUT_C = 32
PAD = 48
NC = 2
NS = 16
NW = NC * NS
EPT = E // NW
CH = 128
NCHUNK = EPT // CH


def _tc_prep(x_ref, w_ref, a_ref, sup_ref, s2_ref):
    sup = jnp.dot(x_ref[...], w_ref[...], preferred_element_type=jnp.float32)
    sup_ref[...] = jnp.concatenate(
        [sup, jnp.zeros((N, PAD - OUT_C), jnp.float32)], axis=1)
    s2_ref[...] = lax.dot_general(
        a_ref[...], sup, (((1,), (1,)), ((), ())),
        preferred_element_type=jnp.float32)


def _sc_edges(sup_hbm, src_hbm, dst_hbm, adj_hbm, s2_hbm, out_hbm,
              acc_sh, ssrc_v, sdst_v, zbuf_v, idxs_v, idxd_v, adj_v, rows_v, sem):
    cid = lax.axis_index("c")
    sid = lax.axis_index("s")
    wid = sid * NC + cid

    zero16 = jnp.zeros((16,), jnp.float32)
    for r in range(CH):
        for k in range(PAD // 16):
            zbuf_v[r, 16 * k:16 * (k + 1)] = zero16
    rows_per_tile = N // NS
    for h in range(rows_per_tile // CH):
        pltpu.sync_copy(zbuf_v, acc_sh.at[pl.ds(sid * rows_per_tile + h * CH, CH)])
    pltpu.sync_copy(s2_hbm.at[0], ssrc_v)
    pltpu.sync_copy(s2_hbm.at[1], sdst_v)
    plsc.subcore_barrier()

    iota16 = lax.iota(jnp.int32, 16)
    col32 = jnp.full((16,), OUT_C, jnp.int32)

    def chunk(c, carry):
        base = wid * EPT + c * CH
        pltpu.sync_copy(src_hbm.at[pl.ds(base, CH)], idxs_v)
        pltpu.sync_copy(dst_hbm.at[pl.ds(base, CH)], idxd_v)
        pltpu.sync_copy(adj_hbm.at[pl.ds(base, CH)], adj_v)
        pltpu.async_copy(sup_hbm.at[idxd_v], rows_v, sem).wait()
        for g in range(CH // 16):
            sl = pl.ds(16 * g, 16)
            ss = plsc.load_gather(ssrc_v, [idxs_v[sl]])
            sd = plsc.load_gather(sdst_v, [idxd_v[sl]])
            s = ss + sd
            w = jnp.exp(jnp.maximum(s, 0.2 * s))
            plsc.store_scatter(rows_v, [iota16 + 16 * g, col32], adj_v[sl])
            for j in range(16):
                e = 16 * g + j
                we = w[j]
                rows_v[e, 0:16] = rows_v[e, 0:16] * we
                rows_v[e, 16:32] = rows_v[e, 16:32] * we
        pltpu.sync_copy(rows_v, acc_sh.at[idxs_v], add=True)
        return carry

    lax.fori_loop(0, NCHUNK, chunk, 0)
    plsc.subcore_barrier()

    for h in range(rows_per_tile // CH):
        off = sid * rows_per_tile + h * CH
        pltpu.sync_copy(acc_sh.at[pl.ds(off, CH)],
                        out_hbm.at[cid, pl.ds(off, CH)])


def _tc_finish(parts_ref, out_ref):
    p = parts_ref[0] + parts_ref[1]
    r = p[:, OUT_C:OUT_C + 1]
    out_ref[...] = p[:, :OUT_C] / jnp.where(r == 0.0, 1.0, r)


@jax.jit
def kernel(x, edge_index, adj_values, weight, attention):
    attn2 = attention.reshape(2, OUT_C)
    sup_pad, s2 = pl.pallas_call(
        _tc_prep,
        out_shape=[
            jax.ShapeDtypeStruct((N, PAD), jnp.float32),
            jax.ShapeDtypeStruct((2, N), jnp.float32),
        ],
    )(x, weight, attn2)

    src = edge_index[0]
    dst = edge_index[1]

    sc_call = functools.partial(
        pl.kernel,
        mesh=plsc.VectorSubcoreMesh(core_axis_name="c", subcore_axis_name="s"),
        out_type=jax.ShapeDtypeStruct((NC, N, PAD), jnp.float32),
        scratch_types=[
            pltpu.VMEM_SHARED((N, PAD), jnp.float32),
            pltpu.VMEM((N,), jnp.float32),
            pltpu.VMEM((N,), jnp.float32),
            pltpu.VMEM((CH, PAD), jnp.float32),
            pltpu.VMEM((CH,), jnp.int32),
            pltpu.VMEM((CH,), jnp.int32),
            pltpu.VMEM((CH,), jnp.float32),
            pltpu.VMEM((CH, PAD), jnp.float32),
            pltpu.SemaphoreType.DMA,
        ],
        compiler_params=pltpu.CompilerParams(
            needs_layout_passes=False, use_tc_tiling_on_sc=False),
    )(_sc_edges)
    parts = sc_call(sup_pad, src, dst, adj_values, s2)

    out = pl.pallas_call(
        _tc_finish,
        out_shape=jax.ShapeDtypeStruct((N, OUT_C), jnp.float32),
    )(parts)
    return out.reshape(N, 1, OUT_C)
